# Initial kernel scaffold; baseline (speedup 1.0000x reference)
import jax
import jax.numpy as jnp
from jax.experimental import pallas as pl


def kernel(x1, x2, e12_type, e12_index):
    raise NotImplementedError("write your pallas kernel here")


# speedup vs baseline: 14.6737x; 14.6737x over previous
import functools

import jax
import jax.numpy as jnp
from jax import lax
from jax.experimental import pallas as pl
from jax.experimental.pallas import tpu as pltpu
from jax.experimental.pallas import tpu_sc as plsc

N_NODES = 50000
N_EDGES = 1600000

NUM_CORES = 2
NUM_SUBCORES = 16
NUM_WORKERS = NUM_CORES * NUM_SUBCORES
EDGES_PER_WORKER = N_EDGES // NUM_WORKERS
CHUNK = 2000
NUM_CHUNKS = EDGES_PER_WORKER // CHUNK
GROUPS = CHUNK // 16

EPS1 = 0.1
EPS2 = 0.1


def _edge_kernel(x1x, x1y, x1z, x2x, x2y, x2z, i0_hbm, i1_hbm, tp_hbm,
                 out_hbm, i0_v, i1_v, tp_v, ax_v, ay_v, az_v, bx_v,
                 by_v, bz_v, acc_v, sem):
    wid = lax.axis_index("s") * NUM_CORES + lax.axis_index("c")
    acc_v[...] = jnp.zeros((16,), jnp.float32)

    def chunk_body(k, carry):
        base = wid * EDGES_PER_WORKER + k * CHUNK
        pltpu.sync_copy(i0_hbm.at[pl.ds(base, CHUNK)], i0_v)
        pltpu.sync_copy(i1_hbm.at[pl.ds(base, CHUNK)], i1_v)
        pltpu.sync_copy(tp_hbm.at[pl.ds(base, CHUNK)], tp_v)
        copies = [
            pltpu.async_copy(x1x.at[i0_v], ax_v, sem),
            pltpu.async_copy(x1y.at[i0_v], ay_v, sem),
            pltpu.async_copy(x1z.at[i0_v], az_v, sem),
            pltpu.async_copy(x2x.at[i1_v], bx_v, sem),
            pltpu.async_copy(x2y.at[i1_v], by_v, sem),
            pltpu.async_copy(x2z.at[i1_v], bz_v, sem),
        ]
        for c in copies:
            c.wait()

        def grp(g, acc):
            s = pl.ds(g * 16, 16)
            dx = ax_v[s] - bx_v[s]
            dy = ay_v[s] - by_v[s]
            dz = az_v[s] - bz_v[s]
            d2 = dx * dx + dy * dy + dz * dz
            seed = jnp.full((16,), 0x5F3759DF, jnp.int32) - (
                lax.bitcast_convert_type(d2, jnp.int32) >> 1)
            y = lax.bitcast_convert_type(seed, jnp.float32)
            h = 0.5 * d2
            y = y * (1.5 - h * y * y)
            y = y * (1.5 - h * y * y)
            y = y * (1.5 - h * y * y)
            d = d2 * y
            t = tp_v[s]
            is0 = t == 0
            le2 = t <= 2
            le4 = t <= 4
            is5 = t == 5
            dmin = jnp.where(
                is0, 0.0,
                jnp.where(le2, 2.8, jnp.where(le4, 2.4,
                                              jnp.where(is5, 2.0, 3.0))))
            dmax = jnp.where(
                is0, 1e30,
                jnp.where(le2, 7.5, jnp.where(le4, 4.1,
                                              jnp.where(is5, 4.0, 7.0))))
            drift = EPS1 * jnp.maximum(d - dmax, 0.0) + EPS2 * jnp.maximum(
                dmin - d, 0.0)
            return acc + drift

        acc_v[...] = lax.fori_loop(0, GROUPS, grp, acc_v[...])
        return carry

    lax.fori_loop(0, NUM_CHUNKS, chunk_body, 0)
    pltpu.sync_copy(acc_v, out_hbm.at[wid])


@jax.jit
def _run(x1x, x1y, x1z, x2x, x2y, x2z, i0, i1, tp):
    call = functools.partial(
        pl.kernel,
        mesh=plsc.VectorSubcoreMesh(core_axis_name="c", subcore_axis_name="s"),
        out_type=jax.ShapeDtypeStruct((NUM_WORKERS, 16), jnp.float32),
        scratch_types=[
            pltpu.VMEM((CHUNK,), jnp.int32),
            pltpu.VMEM((CHUNK,), jnp.int32),
            pltpu.VMEM((CHUNK,), jnp.int32),
            pltpu.VMEM((CHUNK,), jnp.float32),
            pltpu.VMEM((CHUNK,), jnp.float32),
            pltpu.VMEM((CHUNK,), jnp.float32),
            pltpu.VMEM((CHUNK,), jnp.float32),
            pltpu.VMEM((CHUNK,), jnp.float32),
            pltpu.VMEM((CHUNK,), jnp.float32),
            pltpu.VMEM((16,), jnp.float32),
            pltpu.SemaphoreType.DMA,
        ],
    )(_edge_kernel)
    return call(x1x, x1y, x1z, x2x, x2y, x2z, i0, i1, tp)


def kernel(x1, x2, e12_type, e12_index):
    out = _run(x1[:, 0], x1[:, 1], x1[:, 2], x2[:, 0], x2[:, 1], x2[:, 2],
               e12_index[0], e12_index[1], e12_type)
    return jnp.sum(out)
